# baseline (speedup 1.0000x reference)
import functools

import jax
import jax.numpy as jnp
from jax import lax
from jax.experimental import pallas as pl
from jax.experimental.pallas import tpu as pltpu
from jax.experimental.pallas import tpu_sc as plsc

B, L, D = 256, 144, 768
N_PATCH = 576
N = B * L
N_TC = 18432
N_SC = N - N_TC
TC_BLK = 128
V_PAD = 640

NW = 32
ROWS_PER_W = N_SC // NW
C = 32
NCHUNK = ROWS_PER_W // C
UNROLL = 6
NBLK = NCHUNK // UNROLL
LANES = 16

_mesh = plsc.VectorSubcoreMesh(core_axis_name="c", subcore_axis_name="s")


@functools.partial(
    pl.kernel,
    mesh=_mesh,
    out_type=jax.ShapeDtypeStruct((N_SC, D), jnp.float32),
    scratch_types=(
        [pltpu.VMEM((ROWS_PER_W,), jnp.int32)]
        + [pltpu.VMEM((C, D), jnp.float32) for _ in range(5)]
        + [pltpu.SemaphoreType.DMA for _ in range(8)]
    ),
)
def _pe_add_sc(x_hbm, idx_hbm, table_hbm, out_hbm, idx_all,
               xv0, xv1, rv0, rv1, rv2,
               sx0, sx1, sg0, sg1, sg2, so0, so1, so2):
    xv = [xv0, xv1]
    rv = [rv0, rv1, rv2]
    sx = [sx0, sx1]
    sg = [sg0, sg1, sg2]
    so = [so0, so1, so2]

    sid = lax.axis_index("s")
    wid = sid * 2 + lax.axis_index("c")
    base_w = wid * ROWS_PER_W

    pltpu.sync_copy(idx_hbm.at[pl.ds(base_w, ROWS_PER_W)], idx_all)
    for i in range(ROWS_PER_W // LANES):
        sl = pl.ds(i * LANES, LANES)
        idx_all[sl] = idx_all[sl] + 1

    def start_loads(g, bx, br):
        pltpu.async_copy(x_hbm.at[pl.ds(base_w + g * C, C)], xv[bx], sx[bx])
        pltpu.async_copy(table_hbm.at[idx_all.at[pl.ds(g * C, C)]], rv[br], sg[br])

    for g in range(2):
        start_loads(g, g % 2, g % 3)

    def block(blk, carry):
        g0 = blk * UNROLL
        for j in range(UNROLL):
            g = g0 + j
            bx = j % 2
            br = j % 3
            br2 = (j + 2) % 3
            pltpu.make_async_copy(x_hbm.at[pl.ds(base_w, C)], xv[bx], sx[bx]).wait()
            pltpu.make_async_copy(table_hbm.at[pl.ds(0, C)], rv[br], sg[br]).wait()

            @plsc.parallel_loop(0, C, 1, unroll=2)
            def add_row(r):
                for k in range(D // LANES):
                    sl = pl.ds(k * LANES, LANES)
                    plsc.addupdate(rv[br].at[r, sl], xv[bx][r, sl])

            pltpu.async_copy(rv[br], out_hbm.at[pl.ds(base_w + g * C, C)], so[br])

            def drain_prev_scatter():
                pltpu.make_async_copy(
                    rv[br2], out_hbm.at[pl.ds(base_w, C)], so[br2]).wait()

            def prefetch():
                drain_prev_scatter()
                start_loads(g + 2, bx, br2)

            if j == 0:
                pl.when(blk > 0)(drain_prev_scatter)
                start_loads(g + 2, bx, br2)
            elif j < 4:
                prefetch()
            else:
                pl.when(blk < NBLK - 1)(prefetch)
        return carry

    lax.fori_loop(0, NBLK, block, 0)

    for b in range(3):
        pltpu.make_async_copy(rv[b], out_hbm.at[pl.ds(base_w, C)], so[b]).wait()


def _pe_add_tc_body(idx_ref, x_ref, t_ref, o_ref):
    idxv = idx_ref[0, 0, :] + 1
    onehot = (jax.lax.broadcasted_iota(jnp.int32, (TC_BLK, V_PAD), 1)
              == idxv[:, None]).astype(jnp.bfloat16)
    gathered = jnp.dot(onehot, t_ref[...],
                       preferred_element_type=jnp.float32)
    o_ref[...] = x_ref[...] + gathered


_pe_add_tc = pl.pallas_call(
    _pe_add_tc_body,
    grid=(N_TC // TC_BLK,),
    in_specs=[
        pl.BlockSpec((1, 1, TC_BLK), lambda i: (i, 0, 0)),
        pl.BlockSpec((TC_BLK, D), lambda i: (i, 0)),
        pl.BlockSpec((V_PAD, D), lambda i: (0, 0)),
    ],
    out_specs=pl.BlockSpec((TC_BLK, D), lambda i: (i, 0)),
    out_shape=jax.ShapeDtypeStruct((N_TC, D), jnp.float32),
)


def kernel(unmask_patch_embed, unmask_idx, cls_encode, pe_encode):
    del cls_encode
    x = unmask_patch_embed.reshape(N, D)
    idx = unmask_idx.reshape(N).astype(jnp.int32)
    table = pe_encode.reshape(N_PATCH + 1, D)

    idx_tc = idx[:N_TC].reshape(N_TC // TC_BLK, 1, TC_BLK)
    table_bf = jnp.pad(table, ((0, V_PAD - (N_PATCH + 1)), (0, 0))
                       ).astype(jnp.bfloat16)
    out_tc = _pe_add_tc(idx_tc, x[:N_TC], table_bf)

    out_sc = _pe_add_sc(x[N_TC:], idx[N_TC:], table)

    out = jnp.concatenate([out_tc, out_sc], axis=0)
    return out.reshape(B, L, D)


# speedup vs baseline: 2.1967x; 2.1967x over previous
import functools

import jax
import jax.numpy as jnp
from jax import lax
from jax.experimental import pallas as pl
from jax.experimental.pallas import tpu as pltpu
from jax.experimental.pallas import tpu_sc as plsc

B, L, D = 256, 144, 768
N_PATCH = 576
N = B * L
NW = 32
ROWS_PER_W = N // NW
C = 24
NCHUNK = ROWS_PER_W // C
NBUF = 3
NBLK = NCHUNK // NBUF
LANES = 16

_mesh = plsc.VectorSubcoreMesh(core_axis_name="c", subcore_axis_name="s")


@functools.partial(
    pl.kernel,
    mesh=_mesh,
    out_type=jax.ShapeDtypeStruct((N, D), jnp.float32),
    scratch_types=(
        [pltpu.VMEM((ROWS_PER_W,), jnp.int32)]
        + [pltpu.VMEM((C, D), jnp.float32) for _ in range(2 * NBUF)]
        + [pltpu.SemaphoreType.DMA for _ in range(3 * NBUF)]
    ),
)
def _pe_add(x_hbm, idx_hbm, table_hbm, out_hbm, idx_all,
            xv0, xv1, xv2, rv0, rv1, rv2,
            sx0, sx1, sx2, sg0, sg1, sg2, so0, so1, so2):
    xv = [xv0, xv1, xv2]
    rv = [rv0, rv1, rv2]
    sx = [sx0, sx1, sx2]
    sg = [sg0, sg1, sg2]
    so = [so0, so1, so2]

    wid = lax.axis_index("s") * 2 + lax.axis_index("c")
    base_w = wid * ROWS_PER_W

    pltpu.sync_copy(idx_hbm.at[pl.ds(base_w, ROWS_PER_W)], idx_all)
    for i in range(ROWS_PER_W // LANES):
        sl = pl.ds(i * LANES, LANES)
        idx_all[sl] = idx_all[sl] + 1

    def start_loads(g, b):
        pltpu.async_copy(x_hbm.at[pl.ds(base_w + g * C, C)], xv[b], sx[b])
        pltpu.async_copy(table_hbm.at[idx_all.at[pl.ds(g * C, C)]], rv[b], sg[b])

    for g in range(NBUF - 1):
        start_loads(g, g)

    def block(blk, carry):
        g0 = blk * NBUF
        for j in range(NBUF):
            g = g0 + j
            b = j
            b2 = (j + 2) % NBUF
            pltpu.make_async_copy(x_hbm.at[pl.ds(base_w, C)], xv[b], sx[b]).wait()
            pltpu.make_async_copy(x_hbm.at[pl.ds(base_w, C)], rv[b], sg[b]).wait()

            @plsc.parallel_loop(0, C, 1, unroll=2)
            def add_row(r):
                for k in range(D // LANES):
                    sl = pl.ds(k * LANES, LANES)
                    plsc.addupdate(rv[b].at[r, sl], xv[b][r, sl])

            pltpu.async_copy(rv[b], out_hbm.at[pl.ds(base_w + g * C, C)], so[b])

            def drain_prev_scatter():
                pltpu.make_async_copy(
                    rv[b2], out_hbm.at[pl.ds(base_w, C)], so[b2]).wait()

            def prefetch():
                drain_prev_scatter()
                start_loads(g + 2, b2)

            if j == 0:
                pl.when(blk > 0)(drain_prev_scatter)
                start_loads(g + 2, b2)
            else:
                pl.when(blk < NBLK - 1)(prefetch)
        return carry

    lax.fori_loop(0, NBLK, block, 0)

    for b in range(NBUF):
        pltpu.make_async_copy(rv[b], out_hbm.at[pl.ds(base_w, C)], so[b]).wait()


def kernel(unmask_patch_embed, unmask_idx, cls_encode, pe_encode):
    del cls_encode
    x = unmask_patch_embed.reshape(N, D)
    idx = unmask_idx.reshape(N).astype(jnp.int32)
    table = pe_encode.reshape(N_PATCH + 1, D)
    out = _pe_add(x, idx, table)
    return out.reshape(B, L, D)
